# baseline (speedup 1.0000x reference)
import functools

import jax
import jax.numpy as jnp
from jax import lax
from jax.experimental import pallas as pl
from jax.experimental.pallas import tpu as pltpu
from jax.experimental.pallas import tpu_sc as plsc

VOCAB = 1000000
DIM = 64
BATCH = 16384

_NC = 2
_NS = 16
_NW = _NC * _NS
_BPW = BATCH // _NW
_NP = 12
_NR = 8


@functools.partial(
    pl.kernel,
    mesh=plsc.VectorSubcoreMesh(core_axis_name="c", subcore_axis_name="s"),
    out_type=jax.ShapeDtypeStruct((BATCH, DIM), jnp.float32),
    scratch_types=[
        pltpu.VMEM((_BPW,), jnp.int32),
        pltpu.VMEM((_BPW,), jnp.int32),
        pltpu.VMEM((_NP, DIM, 128), jnp.float32),
        pltpu.VMEM((_NR, DIM), jnp.float32),
        pltpu.SMEM((_BPW,), jnp.int32),
        pltpu.SMEM((_BPW,), jnp.int32),
        [pltpu.SemaphoreType.DMA] * _NP,
        pltpu.SemaphoreType.DMA,
    ],
    compiler_params=pltpu.CompilerParams(
        use_tc_tiling_on_sc=True, needs_layout_passes=False
    ),
)
def _sc_gather(sidx_hbm, spos_hbm, tt_hbm, out_hbm, sidx_v, spos_v,
               panels_v, rows_v, cs_s, it_s, psems, osem):
    wid = lax.axis_index("s") * _NC + lax.axis_index("c")
    base = wid * _BPW
    pltpu.sync_copy(sidx_hbm.at[pl.ds(base, _BPW)], sidx_v)
    pltpu.sync_copy(spos_hbm.at[pl.ds(base, _BPW)], spos_v)
    iota = lax.iota(jnp.int32, 16)

    def stage(g, _):
        iv = sidx_v[pl.ds(g * 16, 16)]
        pv = spos_v[pl.ds(g * 16, 16)]
        cvec = iv >> 7
        ivec = (iv & 127) | (pv << 7)
        off = g * 16
        for j in range(16):
            cs_s[off + j] = cvec[j]
            it_s[off + j] = ivec[j]
        return 0

    lax.fori_loop(0, _BPW // 16, stage, 0, unroll=False)

    def fetch(slot, c):
        off = pl.multiple_of(c * 128, 128)
        for s in range(_NP):
            @pl.when(slot == s)
            def _(s=s):
                pltpu.async_copy(
                    tt_hbm.at[:, pl.ds(off, 128)], panels_v.at[s], psems[s]
                )

    fetch(jnp.int32(0), cs_s[0])

    def body(e, carry):
        pf, dpf, dc = carry

        for _ in range(2):
            can_scan = pf < _BPW
            cnew = cs_s[jnp.minimum(pf, _BPW - 1)]
            cprev = cs_s[jnp.minimum(pf - 1, _BPW - 1)]
            is_new = can_scan & (cnew != cprev)
            room = dpf < dc + _NP - 1
            do_issue = is_new & room

            @pl.when(do_issue)
            def _(dpf=dpf, cnew=cnew):
                fetch(dpf % _NP, cnew)

            advance = can_scan & ((~is_new) | room)
            pf = pf + advance.astype(jnp.int32)
            dpf = dpf + do_issue.astype(jnp.int32)

        ce = cs_s[e]
        cprev_e = cs_s[jnp.maximum(e - 1, 0)]
        run_start = (e == 0) | (ce != cprev_e)

        @pl.when(run_start)
        def _():
            slot_new = dc % _NP
            for s in range(_NP):
                @pl.when(slot_new == s)
                def _():
                    pltpu.make_async_copy(
                        tt_hbm.at[:, pl.ds(0, 128)],
                        panels_v.at[s],
                        psems[s],
                    ).wait()

        dc = dc + run_start.astype(jnp.int32)
        slot = (dc - 1) % _NP

        rslot = e % _NR

        @pl.when(e >= _NR)
        def _():
            pltpu.make_async_copy(
                rows_v.at[pl.ds(0, 1)], out_hbm.at[pl.ds(0, 1)], osem
            ).wait()

        item = it_s[e]
        l = item & 127
        pos = item >> 7
        sv = jnp.full((16,), slot, jnp.int32)
        lv = jnp.full((16,), l, jnp.int32)
        rv = jnp.full((16,), rslot, jnp.int32)
        for k in range(DIM // 16):
            fv = iota + (k * 16)
            v = plsc.load_gather(panels_v, [sv, fv, lv])
            plsc.store_scatter(rows_v, [rv, fv], v)

        pltpu.async_copy(
            rows_v.at[pl.ds(rslot, 1)], out_hbm.at[pl.ds(pos, 1)], osem
        )
        return (pf, dpf, dc)

    lax.fori_loop(0, _BPW, body, (jnp.int32(1), jnp.int32(1), jnp.int32(0)),
                  unroll=False)

    def drain(_, x):
        pltpu.make_async_copy(
            rows_v.at[pl.ds(0, 1)], out_hbm.at[pl.ds(0, 1)], osem
        ).wait()
        return x

    lax.fori_loop(0, _NR, drain, 0, unroll=False)


def _mm_body(w_ref, x_ref, b_ref, o_ref):
    o_ref[...] = (
        lax.dot_general(
            w_ref[...],
            x_ref[...],
            (((0,), (1,)), ((), ())),
            preferred_element_type=jnp.float32,
        )
        + b_ref[...]
    )


_BM = 4096


def _tc_linear_t(W, rows, b2d):
    return pl.pallas_call(
        _mm_body,
        grid=(BATCH // _BM,),
        in_specs=[
            pl.BlockSpec((DIM, DIM), lambda i: (0, 0)),
            pl.BlockSpec((_BM, DIM), lambda i: (i, 0)),
            pl.BlockSpec((DIM, 1), lambda i: (0, 0)),
        ],
        out_specs=pl.BlockSpec((DIM, _BM), lambda i: (0, i)),
        out_shape=jax.ShapeDtypeStruct((DIM, BATCH), jnp.float32),
    )(W, rows, b2d)


def kernel(message, table, W_agent, b_agent):
    idx = message.astype(jnp.int32)
    pos = lax.iota(jnp.int32, BATCH)
    sidx, spos = lax.sort([idx, pos], num_keys=1)
    rows = _sc_gather(sidx, spos, table.T)
    return _tc_linear_t(W_agent, rows, b_agent.reshape(DIM, 1)).T


# speedup vs baseline: 1.0623x; 1.0623x over previous
import functools

import jax
import jax.numpy as jnp
from jax import lax
from jax.experimental import pallas as pl
from jax.experimental.pallas import tpu as pltpu
from jax.experimental.pallas import tpu_sc as plsc

VOCAB = 1000000
DIM = 64
BATCH = 16384

_NC = 2
_NS = 16
_NW = _NC * _NS
_BPW = BATCH // _NW
_NP = 8
_NR = 8


@functools.partial(
    pl.kernel,
    mesh=plsc.VectorSubcoreMesh(core_axis_name="c", subcore_axis_name="s"),
    out_type=jax.ShapeDtypeStruct((BATCH, DIM), jnp.float32),
    scratch_types=[
        pltpu.VMEM((_BPW,), jnp.int32),
        pltpu.VMEM((_BPW,), jnp.int32),
        pltpu.VMEM((_NP, DIM, 128), jnp.float32),
        pltpu.VMEM((_NR, DIM), jnp.float32),
        pltpu.SMEM((_BPW,), jnp.int32),
        pltpu.SMEM((_BPW,), jnp.int32),
        [pltpu.SemaphoreType.DMA] * _NP,
        pltpu.SemaphoreType.DMA,
    ],
    compiler_params=pltpu.CompilerParams(
        use_tc_tiling_on_sc=True, needs_layout_passes=False
    ),
)
def _sc_gather(sidx_hbm, spos_hbm, tt_hbm, out_hbm, sidx_v, spos_v,
               panels_v, rows_v, cs_s, it_s, psems, osem):
    wid = lax.axis_index("s") * _NC + lax.axis_index("c")
    base = wid * _BPW
    pltpu.sync_copy(sidx_hbm.at[pl.ds(base, _BPW)], sidx_v)
    pltpu.sync_copy(spos_hbm.at[pl.ds(base, _BPW)], spos_v)
    iota = lax.iota(jnp.int32, 16)

    def stage(g, _):
        iv = sidx_v[pl.ds(g * 16, 16)]
        pv = spos_v[pl.ds(g * 16, 16)]
        cvec = iv >> 7
        ivec = (iv & 127) | (pv << 7)
        off = g * 16
        for j in range(16):
            cs_s[off + j] = cvec[j]
            it_s[off + j] = ivec[j]
        return 0

    lax.fori_loop(0, _BPW // 16, stage, 0, unroll=False)

    def fetch(slot, c):
        off = pl.multiple_of(c * 128, 128)
        for s in range(_NP):
            @pl.when(slot == s)
            def _(s=s):
                pltpu.async_copy(
                    tt_hbm.at[:, pl.ds(off, 128)], panels_v.at[s], psems[s]
                )

    fetch(jnp.int32(0), cs_s[0])

    def body(e, carry):
        pf, dpf, dc = carry

        for _ in range(2):
            can_scan = pf < _BPW
            cnew = cs_s[jnp.minimum(pf, _BPW - 1)]
            cprev = cs_s[jnp.minimum(pf - 1, _BPW - 1)]
            is_new = can_scan & (cnew != cprev)
            room = dpf < dc + _NP - 1
            do_issue = is_new & room

            @pl.when(do_issue)
            def _(dpf=dpf, cnew=cnew):
                fetch(dpf % _NP, cnew)

            advance = can_scan & ((~is_new) | room)
            pf = pf + advance.astype(jnp.int32)
            dpf = dpf + do_issue.astype(jnp.int32)

        ce = cs_s[e]
        cprev_e = cs_s[jnp.maximum(e - 1, 0)]
        run_start = (e == 0) | (ce != cprev_e)

        @pl.when(run_start)
        def _():
            slot_new = dc % _NP
            for s in range(_NP):
                @pl.when(slot_new == s)
                def _():
                    pltpu.make_async_copy(
                        tt_hbm.at[:, pl.ds(0, 128)],
                        panels_v.at[s],
                        psems[s],
                    ).wait()

        dc = dc + run_start.astype(jnp.int32)
        slot = (dc - 1) % _NP

        rslot = e % _NR

        @pl.when(e >= _NR)
        def _():
            pltpu.make_async_copy(
                rows_v.at[pl.ds(0, 1)], out_hbm.at[pl.ds(0, 1)], osem
            ).wait()

        item = it_s[e]
        l = item & 127
        pos = item >> 7
        sv = jnp.full((16,), slot, jnp.int32)
        lv = jnp.full((16,), l, jnp.int32)
        rv = jnp.full((16,), rslot, jnp.int32)
        for k in range(DIM // 16):
            fv = iota + (k * 16)
            v = plsc.load_gather(panels_v, [sv, fv, lv])
            plsc.store_scatter(rows_v, [rv, fv], v)

        pltpu.async_copy(
            rows_v.at[pl.ds(rslot, 1)], out_hbm.at[pl.ds(pos, 1)], osem
        )
        return (pf, dpf, dc)

    lax.fori_loop(0, _BPW, body, (jnp.int32(1), jnp.int32(1), jnp.int32(0)),
                  unroll=False)

    def drain(_, x):
        pltpu.make_async_copy(
            rows_v.at[pl.ds(0, 1)], out_hbm.at[pl.ds(0, 1)], osem
        ).wait()
        return x

    lax.fori_loop(0, _NR, drain, 0, unroll=False)


def _mm_body(w_ref, x_ref, b_ref, o_ref):
    o_ref[...] = (
        lax.dot_general(
            w_ref[...],
            x_ref[...],
            (((0,), (1,)), ((), ())),
            preferred_element_type=jnp.float32,
        )
        + b_ref[...]
    )


_BM = 4096


def _tc_linear_t(W, rows, b2d):
    return pl.pallas_call(
        _mm_body,
        grid=(BATCH // _BM,),
        in_specs=[
            pl.BlockSpec((DIM, DIM), lambda i: (0, 0)),
            pl.BlockSpec((_BM, DIM), lambda i: (i, 0)),
            pl.BlockSpec((DIM, 1), lambda i: (0, 0)),
        ],
        out_specs=pl.BlockSpec((DIM, _BM), lambda i: (0, i)),
        out_shape=jax.ShapeDtypeStruct((DIM, BATCH), jnp.float32),
    )(W, rows, b2d)


def kernel(message, table, W_agent, b_agent):
    idx = message.astype(jnp.int32)
    pos = lax.iota(jnp.int32, BATCH)
    sidx, spos = lax.sort([idx, pos], num_keys=1)
    rows = _sc_gather(sidx, spos, table.T)
    return _tc_linear_t(W_agent, rows, b_agent.reshape(DIM, 1)).T
